# Initial kernel scaffold; baseline (speedup 1.0000x reference)
import functools

import jax
import jax.numpy as jnp
from jax import lax
from jax.experimental import pallas as pl
from jax.experimental.pallas import tpu as pltpu

DIM = 2048
N_HEADS = 32
HEAD_DIM = 128
ROPE_HD = 64
Q_LORA = 1536
SEQ = 2048
EPS = 1e-6
SCALE = HEAD_DIM ** -0.5 * N_HEADS ** -0.5


def _rope(x, cos, sin):
    x1 = x[..., 0::2]
    x2 = x[..., 1::2]
    c = cos[..., 0::2]
    s = sin[..., 0::2]
    o1 = x1 * c - x2 * s
    o2 = x2 * c + x1 * s
    return jnp.stack([o1, o2], axis=-1).reshape(x.shape)


def _k_kernel(x_ref, wk_ref, wp_ref, cos_ref, sin_ref, knw_ref, knb_ref,
              k_ref, w_ref):
    x = x_ref[...]
    kp = jnp.dot(x, wk_ref[...], preferred_element_type=jnp.float32)
    mu = jnp.mean(kp, axis=-1, keepdims=True)
    var = jnp.mean((kp - mu) ** 2, axis=-1, keepdims=True)
    k = (kp - mu) / jnp.sqrt(var + EPS) * knw_ref[...] + knb_ref[...]
    k_pe = _rope(k[:, :ROPE_HD], cos_ref[...], sin_ref[...])
    k_ref[...] = jnp.concatenate([k_pe, k[:, ROPE_HD:]], axis=-1)
    w_ref[...] = jnp.dot(x, wp_ref[...], preferred_element_type=jnp.float32)


def _score_kernel(qr_ref, wqb_ref, k_ref, w_ref, cos_ref, sin_ref, out_ref):
    i = pl.program_id(0)
    blk = qr_ref.shape[0]
    q = jnp.dot(qr_ref[...], wqb_ref[...], preferred_element_type=jnp.float32)
    q = q.reshape(blk, N_HEADS, HEAD_DIM)
    cos = cos_ref[...][:, None, :]
    sin = sin_ref[...][:, None, :]
    q_pe = _rope(q[:, :, :ROPE_HD], cos, sin)
    q = jnp.concatenate([q_pe, q[:, :, ROPE_HD:]], axis=-1)
    w = w_ref[...] * SCALE
    k = k_ref[...]
    acc = jnp.zeros((blk, SEQ), jnp.float32)
    for h in range(N_HEADS):
        l = jax.nn.relu(
            lax.dot_general(q[:, h, :], k, (((1,), (1,)), ((), ())),
                            preferred_element_type=jnp.float32))
        acc = acc + w[:, h:h + 1] * l
    rows = i * blk + lax.broadcasted_iota(jnp.int32, (blk, SEQ), 0)
    cols = lax.broadcasted_iota(jnp.int32, (blk, SEQ), 1)
    out_ref[...] = acc + jnp.where(cols <= rows, 0.0, -1e9).astype(jnp.float32)


def _scores(x, qr, cos, sin, wq_b, wk, weights_proj, k_norm_w, k_norm_b):
    kb = 512
    k_full, w_full = pl.pallas_call(
        _k_kernel,
        grid=(SEQ // kb,),
        in_specs=[
            pl.BlockSpec((kb, DIM), lambda i: (i, 0)),
            pl.BlockSpec((DIM, HEAD_DIM), lambda i: (0, 0)),
            pl.BlockSpec((DIM, N_HEADS), lambda i: (0, 0)),
            pl.BlockSpec((kb, ROPE_HD), lambda i: (i, 0)),
            pl.BlockSpec((kb, ROPE_HD), lambda i: (i, 0)),
            pl.BlockSpec((HEAD_DIM,), lambda i: (0,)),
            pl.BlockSpec((HEAD_DIM,), lambda i: (0,)),
        ],
        out_specs=[
            pl.BlockSpec((kb, HEAD_DIM), lambda i: (i, 0)),
            pl.BlockSpec((kb, N_HEADS), lambda i: (i, 0)),
        ],
        out_shape=[
            jax.ShapeDtypeStruct((SEQ, HEAD_DIM), jnp.float32),
            jax.ShapeDtypeStruct((SEQ, N_HEADS), jnp.float32),
        ],
    )(x, wk, weights_proj, cos, sin, k_norm_w, k_norm_b)

    sb = 256
    scores = pl.pallas_call(
        _score_kernel,
        grid=(SEQ // sb,),
        in_specs=[
            pl.BlockSpec((sb, Q_LORA), lambda i: (i, 0)),
            pl.BlockSpec((Q_LORA, N_HEADS * HEAD_DIM), lambda i: (0, 0)),
            pl.BlockSpec((SEQ, HEAD_DIM), lambda i: (0, 0)),
            pl.BlockSpec((sb, N_HEADS), lambda i: (i, 0)),
            pl.BlockSpec((sb, ROPE_HD), lambda i: (i, 0)),
            pl.BlockSpec((sb, ROPE_HD), lambda i: (i, 0)),
        ],
        out_specs=pl.BlockSpec((sb, SEQ), lambda i: (i, 0)),
        out_shape=jax.ShapeDtypeStruct((SEQ, SEQ), jnp.float32),
    )(qr, wq_b, k_full, w_full, cos, sin)
    return scores


def kernel(x, qr, cos, sin, mask, wq_b, wk, weights_proj, k_norm_w, k_norm_b):
    del mask
    scores = _scores(x[0], qr[0], cos, sin, wq_b, wk, weights_proj,
                     k_norm_w, k_norm_b)
    _, idx = lax.top_k(scores, SEQ)
    return idx[None]


# speedup vs baseline: 1.2075x; 1.2075x over previous
import functools

import jax
import jax.numpy as jnp
from jax import lax
from jax.experimental import pallas as pl
from jax.experimental.pallas import tpu as pltpu

DIM = 2048
N_HEADS = 32
HEAD_DIM = 128
ROPE_HD = 64
Q_LORA = 1536
SEQ = 2048
EPS = 1e-6
SCALE = HEAD_DIM ** -0.5 * N_HEADS ** -0.5


def _rope(x, cos, sin):
    xl = jnp.concatenate([x[..., 1:], x[..., :1]], axis=-1)
    xr = jnp.concatenate([x[..., -1:], x[..., :-1]], axis=-1)
    odd = lax.broadcasted_iota(jnp.int32, x.shape, x.ndim - 1) % 2 == 1
    swapped = jnp.where(odd, xr, xl)
    s_signed = jnp.where(odd, sin, -sin)
    return x * cos + swapped * s_signed


def _k_kernel(x_ref, wk_ref, wp_ref, cos_ref, sin_ref, knw_ref, knb_ref,
              k_ref, w_ref):
    x = x_ref[...]
    kp = jnp.dot(x, wk_ref[...], preferred_element_type=jnp.float32)
    mu = jnp.mean(kp, axis=-1, keepdims=True)
    var = jnp.mean((kp - mu) ** 2, axis=-1, keepdims=True)
    k = (kp - mu) / jnp.sqrt(var + EPS) * knw_ref[...] + knb_ref[...]
    k_pe = _rope(k[:, :ROPE_HD], cos_ref[...], sin_ref[...])
    k_ref[...] = jnp.concatenate([k_pe, k[:, ROPE_HD:]], axis=-1)
    w_ref[...] = jnp.dot(x, wp_ref[...], preferred_element_type=jnp.float32)


def _score_kernel(qr_ref, wqb_ref, k_ref, w_ref, cos_ref, sin_ref, out_ref):
    i = pl.program_id(0)
    blk = qr_ref.shape[0]
    q = jnp.dot(qr_ref[...], wqb_ref[...], preferred_element_type=jnp.float32)
    q = q.reshape(blk, N_HEADS, HEAD_DIM)
    cos = cos_ref[...][:, None, :]
    sin = sin_ref[...][:, None, :]
    q_pe = _rope(q[:, :, :ROPE_HD], cos, sin)
    q = jnp.concatenate([q_pe, q[:, :, ROPE_HD:]], axis=-1)
    w = w_ref[...] * SCALE
    k = k_ref[...]
    acc = jnp.zeros((blk, SEQ), jnp.float32)
    for h in range(N_HEADS):
        l = jax.nn.relu(
            lax.dot_general(q[:, h, :], k, (((1,), (1,)), ((), ())),
                            preferred_element_type=jnp.float32))
        acc = acc + w[:, h:h + 1] * l
    rows = i * blk + lax.broadcasted_iota(jnp.int32, (blk, SEQ), 0)
    cols = lax.broadcasted_iota(jnp.int32, (blk, SEQ), 1)
    out_ref[...] = acc + jnp.where(cols <= rows, 0.0, -1e9).astype(jnp.float32)


def _scores(x, qr, cos, sin, wq_b, wk, weights_proj, k_norm_w, k_norm_b):
    kb = 512
    k_full, w_full = pl.pallas_call(
        _k_kernel,
        grid=(SEQ // kb,),
        in_specs=[
            pl.BlockSpec((kb, DIM), lambda i: (i, 0)),
            pl.BlockSpec((DIM, HEAD_DIM), lambda i: (0, 0)),
            pl.BlockSpec((DIM, N_HEADS), lambda i: (0, 0)),
            pl.BlockSpec((kb, ROPE_HD), lambda i: (i, 0)),
            pl.BlockSpec((kb, ROPE_HD), lambda i: (i, 0)),
            pl.BlockSpec((HEAD_DIM,), lambda i: (0,)),
            pl.BlockSpec((HEAD_DIM,), lambda i: (0,)),
        ],
        out_specs=[
            pl.BlockSpec((kb, HEAD_DIM), lambda i: (i, 0)),
            pl.BlockSpec((kb, N_HEADS), lambda i: (i, 0)),
        ],
        out_shape=[
            jax.ShapeDtypeStruct((SEQ, HEAD_DIM), jnp.float32),
            jax.ShapeDtypeStruct((SEQ, N_HEADS), jnp.float32),
        ],
    )(x, wk, weights_proj, cos, sin, k_norm_w, k_norm_b)

    sb = 256
    scores = pl.pallas_call(
        _score_kernel,
        grid=(SEQ // sb,),
        in_specs=[
            pl.BlockSpec((sb, Q_LORA), lambda i: (i, 0)),
            pl.BlockSpec((Q_LORA, N_HEADS * HEAD_DIM), lambda i: (0, 0)),
            pl.BlockSpec((SEQ, HEAD_DIM), lambda i: (0, 0)),
            pl.BlockSpec((sb, N_HEADS), lambda i: (i, 0)),
            pl.BlockSpec((sb, ROPE_HD), lambda i: (i, 0)),
            pl.BlockSpec((sb, ROPE_HD), lambda i: (i, 0)),
        ],
        out_specs=pl.BlockSpec((sb, SEQ), lambda i: (i, 0)),
        out_shape=jax.ShapeDtypeStruct((SEQ, SEQ), jnp.float32),
    )(qr, wq_b, k_full, w_full, cos, sin)
    return scores


def kernel(x, qr, cos, sin, mask, wq_b, wk, weights_proj, k_norm_w, k_norm_b):
    del mask
    scores = _scores(x[0], qr[0], cos, sin, wq_b, wk, weights_proj,
                     k_norm_w, k_norm_b)
    _, idx = lax.top_k(scores, SEQ)
    return idx[None]
